# speedup vs baseline: 1.2359x; 1.2359x over previous
import functools

import jax
import jax.numpy as jnp
from jax import lax
from jax.experimental import pallas as pl
from jax.experimental.pallas import tpu as pltpu
from jax.experimental.pallas import tpu_sc as plsc


def _make_sc_embed(n_rows: int, t_len: int, d: int):
    info = plsc.get_sparse_core_info()
    nc, ns, nl = info.num_cores, info.num_subcores, info.num_lanes
    nw = nc * ns
    assert n_rows % nw == 0
    rpw = n_rows // nw
    chunk = 128
    assert rpw % chunk == 0
    nch = rpw // chunk
    assert d % nl == 0
    mesh = plsc.VectorSubcoreMesh(core_axis_name="c", subcore_axis_name="s")

    @functools.partial(
        pl.kernel,
        mesh=mesh,
        out_type=jax.ShapeDtypeStruct((n_rows, d), jnp.float32),
        scratch_types=[
            pltpu.VMEM((nch, chunk), jnp.int32),
            pltpu.VMEM((nch, chunk, d), jnp.float32),
            pltpu.VMEM((nch, chunk, d), jnp.float32),
            pltpu.SemaphoreType.DMA,
        ],
    )
    def k(x_hbm, table_hbm, pos_hbm, out_hbm, idx_v, rows_v, pos_v, sem):
        wid = lax.axis_index("s") * nc + lax.axis_index("c")
        base = wid * rpw
        t0 = lax.rem(base, t_len)
        copies = []
        for j in range(nch):
            pltpu.sync_copy(x_hbm.at[pl.ds(base + j * chunk, chunk)],
                            idx_v.at[j])
            copies.append(
                pltpu.async_copy(table_hbm.at[idx_v.at[j]], rows_v.at[j], sem))
        for j in range(nch):
            pltpu.sync_copy(pos_hbm.at[pl.ds(t0 + j * chunk, chunk)],
                            pos_v.at[j])
        for cp in copies:
            cp.wait()

        def add_row(r, _):
            for j in range(nch):
                for cidx in range(d // nl):
                    sl = pl.ds(cidx * nl, nl)
                    plsc.addupdate(rows_v.at[j, r, sl], pos_v[j, r, sl])
            return 0

        lax.fori_loop(0, chunk, add_row, 0)

        for j in range(nch):
            pltpu.sync_copy(rows_v.at[j],
                            out_hbm.at[pl.ds(base + j * chunk, chunk)])

    return k


def kernel(x, embedding, positional_encoding):
    b, t = x.shape
    v, d = embedding.shape
    xf = x.reshape(b * t).astype(jnp.int32)
    fn = _make_sc_embed(b * t, t, d)
    out = fn(xf, embedding, positional_encoding)
    return out.reshape(b, t, d)


# speedup vs baseline: 1.3482x; 1.0908x over previous
import functools

import jax
import jax.numpy as jnp
from jax import lax
from jax.experimental import pallas as pl
from jax.experimental.pallas import tpu as pltpu
from jax.experimental.pallas import tpu_sc as plsc


def _make_sc_embed(b_sz: int, t_len: int, d: int):
    info = plsc.get_sparse_core_info()
    nc, ns, nl = info.num_cores, info.num_subcores, info.num_lanes
    nw = nc * ns
    assert t_len % nw == 0
    tpw = t_len // nw
    assert tpw <= 128
    assert tpw % 8 == 0
    assert d % nl == 0
    nvec = d // nl
    mesh = plsc.VectorSubcoreMesh(core_axis_name="c", subcore_axis_name="s")

    @functools.partial(
        pl.kernel,
        mesh=mesh,
        out_type=jax.ShapeDtypeStruct((b_sz, t_len, d), jnp.float32),
        scratch_types=[
            pltpu.VMEM((b_sz, tpw), jnp.int32),
            pltpu.VMEM((b_sz, tpw, d), jnp.float32),
            pltpu.VMEM((tpw, d), jnp.float32),
            pltpu.SemaphoreType.DMA,
            pltpu.SemaphoreType.DMA,
        ],
    )
    def k(x_hbm, table_hbm, pos_hbm, out_hbm, idx_v, rows_v, pos_v, gsem, ssem):
        wid = lax.axis_index("s") * nc + lax.axis_index("c")
        t0 = wid * tpw
        gathers = []
        for b in range(b_sz):
            pltpu.sync_copy(x_hbm.at[b, pl.ds(t0, tpw)], idx_v.at[b])
            gathers.append(
                pltpu.async_copy(table_hbm.at[idx_v.at[b]], rows_v.at[b], gsem))
        pltpu.sync_copy(pos_hbm.at[pl.ds(t0, tpw)], pos_v)

        stores = []
        for b in range(b_sz):
            gathers[b].wait()

            def add_row(r, _, b=b):
                for c in range(nvec):
                    sl = pl.ds(c * nl, nl)
                    plsc.addupdate(rows_v.at[b, r, sl], pos_v[r, sl])
                return 0

            lax.fori_loop(0, tpw, add_row, 0)
            stores.append(
                pltpu.async_copy(rows_v.at[b], out_hbm.at[b, pl.ds(t0, tpw)],
                                 ssem))
        for st in stores:
            st.wait()

    return k


def kernel(x, embedding, positional_encoding):
    b, t = x.shape
    v, d = embedding.shape
    fn = _make_sc_embed(b, t, d)
    return fn(x.astype(jnp.int32), embedding, positional_encoding)


# speedup vs baseline: 1.4053x; 1.0424x over previous
import functools

import jax
import jax.numpy as jnp
from jax import lax
from jax.experimental import pallas as pl
from jax.experimental.pallas import tpu as pltpu
from jax.experimental.pallas import tpu_sc as plsc


def _make_sc_embed(b_sz: int, t_len: int, d: int):
    info = plsc.get_sparse_core_info()
    nc, ns, nl = info.num_cores, info.num_subcores, info.num_lanes
    nw = nc * ns
    assert t_len % nw == 0
    tpw = t_len // nw
    assert b_sz % 2 == 0
    npair = b_sz // 2
    assert 2 * tpw <= 128
    assert tpw % 8 == 0
    assert d % nl == 0
    nvec = d // nl
    mesh = plsc.VectorSubcoreMesh(core_axis_name="c", subcore_axis_name="s")

    @functools.partial(
        pl.kernel,
        mesh=mesh,
        out_type=jax.ShapeDtypeStruct((b_sz, t_len, d), jnp.float32),
        scratch_types=[
            pltpu.VMEM((npair, 2 * tpw), jnp.int32),
            pltpu.VMEM((npair, 2 * tpw, d), jnp.float32),
            pltpu.VMEM((tpw, d), jnp.float32),
            pltpu.SemaphoreType.DMA,
            pltpu.SemaphoreType.DMA,
            pltpu.SemaphoreType.DMA,
        ],
    )
    def k(x_hbm, table_hbm, pos_hbm, out_hbm, idx_v, rows_v, pos_v,
          isem, gsem, ssem):
        wid = lax.axis_index("s") * nc + lax.axis_index("c")
        t0 = wid * tpw
        idx_copies = []
        for j in range(npair):
            for h in range(2):
                idx_copies.append(pltpu.async_copy(
                    x_hbm.at[2 * j + h, pl.ds(t0, tpw)],
                    idx_v.at[j, pl.ds(h * tpw, tpw)], isem))
        pltpu.sync_copy(pos_hbm.at[pl.ds(t0, tpw)], pos_v)

        gathers = []
        for j in range(npair):
            idx_copies[2 * j].wait()
            idx_copies[2 * j + 1].wait()
            gathers.append(
                pltpu.async_copy(table_hbm.at[idx_v.at[j]], rows_v.at[j],
                                 gsem))

        stores = []
        for j in range(npair):
            gathers[j].wait()

            def add_row(r, _, j=j):
                for c in range(nvec):
                    sl = pl.ds(c * nl, nl)
                    v = pos_v[r, sl]
                    plsc.addupdate(rows_v.at[j, r, sl], v)
                    plsc.addupdate(rows_v.at[j, tpw + r, sl], v)
                return 0

            lax.fori_loop(0, tpw, add_row, 0)
            for h in range(2):
                stores.append(pltpu.async_copy(
                    rows_v.at[j, pl.ds(h * tpw, tpw)],
                    out_hbm.at[2 * j + h, pl.ds(t0, tpw)], ssem))
        for st in stores:
            st.wait()

    return k


def kernel(x, embedding, positional_encoding):
    b, t = x.shape
    v, d = embedding.shape
    fn = _make_sc_embed(b, t, d)
    return fn(x.astype(jnp.int32), embedding, positional_encoding)
